# baseline (speedup 1.0000x reference)
import functools

import jax
import jax.numpy as jnp
from jax import lax
from jax.experimental import pallas as pl
from jax.experimental.pallas import tpu as pltpu
from jax.experimental.pallas import tpu_sc as plsc

VOCAB = 51200
DIM = 2048
BATCH = 4
SEQ = 8192
TOKENS = BATCH * SEQ

NC = 2
NS = 16
NW = NC * NS
B_PER_W = TOKENS // NW
CA = 32
CB = 24
PERIOD = CA + CB
NPAIR = 18
TAIL = B_PER_W - NPAIR * PERIOD
IDX_PAD = (NPAIR + 1) * PERIOD - CB

_MESH = plsc.VectorSubcoreMesh(core_axis_name="c", subcore_axis_name="s")


@functools.partial(
    pl.kernel,
    out_type=jax.ShapeDtypeStruct((TOKENS, DIM), jnp.float32),
    mesh=_MESH,
    scratch_types=[
        pltpu.VMEM((IDX_PAD,), jnp.int32),
        pltpu.VMEM((CA, DIM), jnp.float32),
        pltpu.VMEM((CB, DIM), jnp.float32),
        pltpu.SemaphoreType.DMA,
        pltpu.SemaphoreType.DMA,
    ],
)
def _embed_sc(idx_hbm, table_hbm, out_hbm, idx_v, buf0, buf1, gsem0, gsem1):
    wid = lax.axis_index("s") * NC + lax.axis_index("c")
    base = wid * B_PER_W
    pltpu.sync_copy(idx_hbm.at[wid], idx_v.at[pl.ds(0, B_PER_W)])
    idx_v[pl.ds(B_PER_W, 16)] = jnp.zeros((16,), jnp.int32)

    def fire_a(i, sem):
        pltpu.async_copy(
            table_hbm.at[idx_v.at[pl.ds(i * PERIOD, CA)]], buf0, sem)

    def wait_a(i, sem):
        pltpu.make_async_copy(
            table_hbm.at[idx_v.at[pl.ds(i * PERIOD, CA)]], buf0, sem).wait()

    def fire_b(i, sem):
        pltpu.async_copy(
            table_hbm.at[idx_v.at[pl.ds(i * PERIOD + CA, CB)]], buf1, sem)

    def wait_b(i, sem):
        pltpu.make_async_copy(
            table_hbm.at[idx_v.at[pl.ds(i * PERIOD + CA, CB)]], buf1,
            sem).wait()

    fire_a(0, gsem0)

    def pair_body(i, carry):
        fire_b(i, gsem1)
        wait_a(i, gsem0)
        pltpu.sync_copy(buf0, out_hbm.at[pl.ds(base + i * PERIOD, CA)])
        fire_a(i + 1, gsem0)
        wait_b(i, gsem1)
        pltpu.sync_copy(buf1, out_hbm.at[pl.ds(base + i * PERIOD + CA, CB)])
        return carry

    lax.fori_loop(0, NPAIR, pair_body, 0)

    wait_a(NPAIR, gsem0)
    pltpu.sync_copy(
        buf0.at[pl.ds(0, TAIL)],
        out_hbm.at[pl.ds(base + NPAIR * PERIOD, TAIL)])


def kernel(input_ids, table):
    idx = input_ids.reshape(NW, B_PER_W).astype(jnp.int32)
    out = _embed_sc(idx, table)
    return out.reshape(BATCH, SEQ, DIM)


# speedup vs baseline: 1.1790x; 1.1790x over previous
import functools

import jax
import jax.numpy as jnp
from jax import lax
from jax.experimental import pallas as pl
from jax.experimental.pallas import tpu as pltpu
from jax.experimental.pallas import tpu_sc as plsc

VOCAB = 51200
DIM = 2048
BATCH = 4
SEQ = 8192
TOKENS = BATCH * SEQ

NC = 2
NS = 16
NW = NC * NS
B_PER_W = TOKENS // NW
PHASE = 48
NPHASE = B_PER_W // PHASE
TAIL = B_PER_W - NPHASE * PHASE

_MESH = plsc.VectorSubcoreMesh(core_axis_name="c", subcore_axis_name="s")


@functools.partial(
    pl.kernel,
    out_type=jax.ShapeDtypeStruct((TOKENS, DIM), jnp.float32),
    mesh=_MESH,
    scratch_types=[
        pltpu.VMEM((B_PER_W,), jnp.int32),
        pltpu.VMEM((PHASE, DIM), jnp.float32),
        pltpu.SemaphoreType.DMA,
    ],
)
def _embed_sc(idx_hbm, table_hbm, out_hbm, idx_v, buf, gsem):
    wid = lax.axis_index("s") * NC + lax.axis_index("c")
    base = wid * B_PER_W
    pltpu.sync_copy(idx_hbm.at[wid], idx_v)

    def phase_body(i, carry):
        off = i * PHASE
        pltpu.async_copy(
            table_hbm.at[idx_v.at[pl.ds(off, PHASE)]], buf, gsem).wait()
        pltpu.sync_copy(buf, out_hbm.at[pl.ds(base + off, PHASE)])
        return carry

    lax.fori_loop(0, NPHASE, phase_body, 0)

    off = NPHASE * PHASE
    pltpu.async_copy(
        table_hbm.at[idx_v.at[pl.ds(off, TAIL)]],
        buf.at[pl.ds(0, TAIL)], gsem).wait()
    pltpu.sync_copy(
        buf.at[pl.ds(0, TAIL)], out_hbm.at[pl.ds(base + off, TAIL)])


def kernel(input_ids, table):
    idx = input_ids.reshape(NW, B_PER_W).astype(jnp.int32)
    out = _embed_sc(idx, table)
    return out.reshape(BATCH, SEQ, DIM)


# speedup vs baseline: 1.1941x; 1.0127x over previous
import functools

import jax
import jax.numpy as jnp
from jax import lax
from jax.experimental import pallas as pl
from jax.experimental.pallas import tpu as pltpu
from jax.experimental.pallas import tpu_sc as plsc

VOCAB = 51200
DIM = 2048
BATCH = 4
SEQ = 8192
TOKENS = BATCH * SEQ

NC = 2
NS = 16
NW = NC * NS
B_PER_W = TOKENS // NW
PHASE = 56
NPHASE = B_PER_W // PHASE
TAIL = B_PER_W - NPHASE * PHASE

_MESH = plsc.VectorSubcoreMesh(core_axis_name="c", subcore_axis_name="s")


@functools.partial(
    pl.kernel,
    out_type=jax.ShapeDtypeStruct((TOKENS, DIM), jnp.float32),
    mesh=_MESH,
    scratch_types=[
        pltpu.VMEM((B_PER_W,), jnp.int32),
        pltpu.VMEM((PHASE, DIM), jnp.float32),
        pltpu.SemaphoreType.DMA,
    ],
)
def _embed_sc(idx_hbm, table_hbm, out_hbm, idx_v, buf, gsem):
    wid = lax.axis_index("s") * NC + lax.axis_index("c")
    base = wid * B_PER_W
    pltpu.sync_copy(idx_hbm.at[wid], idx_v)

    def phase_body(i, carry):
        off = i * PHASE
        pltpu.async_copy(
            table_hbm.at[idx_v.at[pl.ds(off, PHASE)]], buf, gsem).wait()
        pltpu.sync_copy(buf, out_hbm.at[pl.ds(base + off, PHASE)])
        return carry

    lax.fori_loop(0, NPHASE, phase_body, 0)

    off = NPHASE * PHASE
    pltpu.async_copy(
        table_hbm.at[idx_v.at[pl.ds(off, TAIL)]],
        buf.at[pl.ds(0, TAIL)], gsem).wait()
    pltpu.sync_copy(
        buf.at[pl.ds(0, TAIL)], out_hbm.at[pl.ds(base + off, TAIL)])


def kernel(input_ids, table):
    idx = input_ids.reshape(NW, B_PER_W).astype(jnp.int32)
    out = _embed_sc(idx, table)
    return out.reshape(BATCH, SEQ, DIM)
